# baseline (speedup 1.0000x reference)
import dataclasses
import functools
import math

import jax
import jax.numpy as jnp
from jax import lax
from jax.experimental import pallas as pl
from jax.experimental.pallas import tpu as pltpu
from jax.experimental.pallas import tpu_sc as plsc

N = 10000
E = 160000
F = 256
HF = 128
SCALE = math.sqrt(F)

NPAD = 10240
EPAD = 163840
NC = 2
NS = 16
PACK_SHIFT = 14

E_TILE1 = EPAD // (NC * NS)
C1 = 128
P1_CHUNKS = E_TILE1 // C1

E_TILE2 = EPAD // NS
C2 = 32
P2_CHUNKS = E_TILE2 // C2
ROWS_PER_TILE = NPAD // NS

_mesh = plsc.VectorSubcoreMesh(core_axis_name="c", subcore_axis_name="s")

_sc_params = pltpu.CompilerParams()
if "needs_layout_passes" in pltpu.CompilerParams.__dataclass_fields__:
    _sc_params = dataclasses.replace(_sc_params, needs_layout_passes=False)


def _unpack_idx(packed_all, off, ridx, cidx, n):
    @pl.loop(0, n // 16)
    def _grp(g):
        p = packed_all[pl.ds(off + g * 16, 16)]
        ridx[pl.ds(g * 16, 16)] = lax.shift_right_logical(p, PACK_SHIFT)
        cidx[pl.ds(g * 16, 16)] = lax.bitwise_and(p, (1 << PACK_SHIFT) - 1)


def _pack_words(lo_f32, hi_f32):
    lo = lax.bitcast_convert_type(lo_f32.astype(jnp.bfloat16), jnp.uint16)
    hi = lax.bitcast_convert_type(hi_f32.astype(jnp.bfloat16), jnp.uint16)
    return lo.astype(jnp.int32) | (hi.astype(jnp.int32) << 16)


def _proj_body(x_ref, wq_ref, wk_ref, qw_ref, kw_ref, q0w_ref, q1w_ref):
    xb = x_ref[...]
    dn = (((1,), (1,)), ((), ()))
    q = lax.dot_general(xb, wq_ref[...], dn,
                        preferred_element_type=jnp.float32,
                        precision=lax.Precision.HIGHEST)
    k = lax.dot_general(xb, wk_ref[...], dn,
                        preferred_element_type=jnp.float32,
                        precision=lax.Precision.HIGHEST)
    qw_ref[...] = _pack_words(q[:, :HF], q[:, HF:])
    kw_ref[...] = _pack_words(k[:, :HF], k[:, HF:])
    q0w_ref[...] = q[:, :HF]
    q1w_ref[...] = q[:, HF:]


def _proj(x_pad, Wq, Wk):
    blk = 1024
    wordt = jax.ShapeDtypeStruct((NPAD, HF), jnp.int32)
    half = jax.ShapeDtypeStruct((NPAD, HF), jnp.float32)
    return pl.pallas_call(
        _proj_body,
        grid=(NPAD // blk,),
        in_specs=[
            pl.BlockSpec((blk, F), lambda i: (i, 0)),
            pl.BlockSpec((F, F), lambda i: (0, 0)),
            pl.BlockSpec((F, F), lambda i: (0, 0)),
        ],
        out_specs=[
            pl.BlockSpec((blk, HF), lambda i: (i, 0)),
            pl.BlockSpec((blk, HF), lambda i: (i, 0)),
            pl.BlockSpec((blk, HF), lambda i: (i, 0)),
            pl.BlockSpec((blk, HF), lambda i: (i, 0)),
        ],
        out_shape=[wordt, wordt, half, half],
    )(x_pad, Wq, Wk)


def _p1_start(q_hbm, k_hbm, packed, off, ridx, cidx, qv, kv, sem):
    _unpack_idx(packed, off, ridx, cidx, C1)
    pltpu.async_copy(q_hbm.at[ridx], qv, sem)
    pltpu.async_copy(k_hbm.at[cidx], kv, sem)


def _p1_wait(q_hbm, k_hbm, ridx, cidx, qv, kv, sem):
    pltpu.make_async_copy(q_hbm.at[ridx], qv, sem).wait()
    pltpu.make_async_copy(k_hbm.at[cidx], kv, sem).wait()


def _bf16_pair(ref, e, t):
    words = ref[e, pl.ds(t * 16, 16)]
    hi = plsc.bitcast(lax.bitwise_and(words, jnp.int32(-65536)), jnp.float32)
    lo = plsc.bitcast(lax.shift_left(words, 16), jnp.float32)
    return lo, hi


def _p1_compute(qv, kv, wv_all, off):
    lane = lax.iota(jnp.int32, 16)

    @pl.loop(0, C1 // 16)
    def _grp(g):
        dots = jnp.zeros((16,), jnp.float32)
        for j in range(16):
            e = g * 16 + j
            acc0 = jnp.zeros((16,), jnp.float32)
            acc1 = jnp.zeros((16,), jnp.float32)
            for t in range(F // 32):
                qlo, qhi = _bf16_pair(qv, e, t)
                klo, khi = _bf16_pair(kv, e, t)
                acc0 = acc0 + qlo * klo
                acc1 = acc1 + qhi * khi
            dots = jnp.where(lane == j, jnp.sum(acc0 + acc1), dots)
        wv_all[pl.ds(off + g * 16, 16)] = jnp.exp(dots * (1.0 / SCALE))


def _pass1_body(q_hbm, k_hbm, packed_hbm, w_hbm,
                packed, wv_all, ridx0, cidx0, ridx1, cidx1,
                qv0, kv0, qv1, kv1, sem0, sem1):
    wid = lax.axis_index("s") * NC + lax.axis_index("c")
    base = wid * E_TILE1
    npairs = P1_CHUNKS // 2

    pltpu.sync_copy(packed_hbm.at[pl.ds(base, E_TILE1)], packed)
    _p1_start(q_hbm, k_hbm, packed, 0, ridx0, cidx0, qv0, kv0, sem0)

    @pl.loop(0, npairs)
    def _pair(i):
        off0 = (2 * i) * C1
        _p1_start(q_hbm, k_hbm, packed, off0 + C1, ridx1, cidx1, qv1, kv1, sem1)
        _p1_wait(q_hbm, k_hbm, ridx0, cidx0, qv0, kv0, sem0)
        _p1_compute(qv0, kv0, wv_all, off0)

        @pl.when(i < npairs - 1)
        def _pref():
            _p1_start(q_hbm, k_hbm, packed, off0 + 2 * C1,
                      ridx0, cidx0, qv0, kv0, sem0)

        _p1_wait(q_hbm, k_hbm, ridx1, cidx1, qv1, kv1, sem1)
        _p1_compute(qv1, kv1, wv_all, off0 + C1)

    pltpu.sync_copy(wv_all, w_hbm.at[pl.ds(base, E_TILE1)])


def _pass1(q, k, packed_p):
    kfn = pl.kernel(
        _pass1_body,
        out_type=jax.ShapeDtypeStruct((EPAD,), jnp.float32),
        mesh=_mesh,
        compiler_params=_sc_params,
        scratch_types=[
            pltpu.VMEM((E_TILE1,), jnp.int32),
            pltpu.VMEM((E_TILE1,), jnp.float32),
            pltpu.VMEM((C1,), jnp.int32),
            pltpu.VMEM((C1,), jnp.int32),
            pltpu.VMEM((C1,), jnp.int32),
            pltpu.VMEM((C1,), jnp.int32),
            pltpu.VMEM((C1, HF), jnp.int32),
            pltpu.VMEM((C1, HF), jnp.int32),
            pltpu.VMEM((C1, HF), jnp.int32),
            pltpu.VMEM((C1, HF), jnp.int32),
            pltpu.SemaphoreType.DMA,
            pltpu.SemaphoreType.DMA,
        ],
    )
    return kfn(q, k, packed_p)


def _p2_scale(qv, wv_all, off):
    @pl.loop(0, C2 // 16)
    def _grp(g):
        ws = wv_all[pl.ds(off + g * 16, 16)]
        for j in range(16):
            e = g * 16 + j
            we = ws[j]
            for t in range(HF // 16):
                qv[e, pl.ds(t * 16, 16)] = qv[e, pl.ds(t * 16, 16)] * we


def _p2_scatter_start(qv, wv_all, off, cidx, snum, sden, sem):
    pltpu.async_copy(qv, snum.at[cidx], sem, add=True)
    pltpu.async_copy(wv_all.at[pl.ds(off, C2)], sden.at[cidx], sem, add=True)


def _p2_scatter_wait(qv, wv_all, off, cidx, snum, sden, sem):
    pltpu.make_async_copy(qv, snum.at[cidx], sem).wait()
    pltpu.make_async_copy(wv_all.at[pl.ds(off, C2)], sden.at[cidx], sem).wait()


def _pass2_body(q0_hbm, q1_hbm, packed_hbm, w_hbm,
                num0_hbm, num1_hbm, den_hbm,
                packed, wv_all,
                ra, ca, rb, cb, rc, cc, rd, cd,
                qva, qvb, qvc, qvd, zd,
                snum, sden,
                gsa, gsb, gsc, gsd, ssa, ssb, ssc, ssd):
    cid = lax.axis_index("c")
    sid = lax.axis_index("s")
    base = sid * E_TILE2
    ridx = [ra, rb, rc, rd]
    cidx = [ca, cb, cc, cd]
    qv = [qva, qvb, qvc, qvd]
    gsem = [gsa, gsb, gsc, gsd]
    ssem = [ssa, ssb, ssc, ssd]

    pltpu.sync_copy(packed_hbm.at[pl.ds(base, E_TILE2)], packed)
    pltpu.sync_copy(w_hbm.at[pl.ds(base, E_TILE2)], wv_all)

    @pl.loop(0, C2)
    def _z(r):
        for t in range(HF // 16):
            qva[r, pl.ds(t * 16, 16)] = jnp.zeros((16,), jnp.float32)

    @pl.loop(0, ROWS_PER_TILE // 16)
    def _zd(i):
        zd[pl.ds(i * 16, 16)] = jnp.zeros((16,), jnp.float32)

    @pl.loop(0, ROWS_PER_TILE // C2)
    def _zs(b):
        pltpu.sync_copy(qva, snum.at[pl.ds(sid * ROWS_PER_TILE + b * C2, C2)])

    pltpu.sync_copy(zd, sden.at[pl.ds(sid * ROWS_PER_TILE, ROWS_PER_TILE)])
    plsc.subcore_barrier()

    nch = P2_CHUNKS

    def run(q_hbm):
        for p in (0, 1):
            _unpack_idx(packed, p * C2, ridx[p], cidx[p], C2)
            pltpu.async_copy(q_hbm.at[ridx[p]], qv[p], gsem[p])

        @pl.loop(0, nch // 4)
        def _it(i):
            cbase = 4 * i
            for p in range(4):
                c = cbase + p
                off = c * C2
                pltpu.make_async_copy(q_hbm.at[ridx[p]], qv[p], gsem[p]).wait()
                _p2_scale(qv[p], wv_all, off)
                _p2_scatter_start(qv[p], wv_all, off, cidx[p], snum, sden,
                                  ssem[p])
                s2 = (p + 2) % 4

                @pl.when(c + 2 < nch)
                def _pf():
                    @pl.when(c >= 2)
                    def _ws():
                        _p2_scatter_wait(qv[s2], wv_all, (c - 2) * C2,
                                         cidx[s2], snum, sden, ssem[s2])

                    _unpack_idx(packed, (c + 2) * C2, ridx[s2], cidx[s2], C2)
                    pltpu.async_copy(q_hbm.at[ridx[s2]], qv[s2], gsem[s2])

        for p in range(4):
            _p2_scatter_wait(qv[p], wv_all, (nch - 4 + p) * C2, cidx[p],
                             snum, sden, ssem[p])

    @pl.when(cid == 0)
    def _c0():
        run(q0_hbm)

    @pl.when(cid == 1)
    def _c1():
        run(q1_hbm)

    plsc.subcore_barrier()

    @pl.loop(0, ROWS_PER_TILE // C2)
    def _wb(b):
        r0 = sid * ROWS_PER_TILE + b * C2

        @pl.when(cid == 0)
        def _w0():
            pltpu.sync_copy(snum.at[pl.ds(r0, C2)], num0_hbm.at[pl.ds(r0, C2)])

        @pl.when(cid == 1)
        def _w1():
            pltpu.sync_copy(snum.at[pl.ds(r0, C2)], num1_hbm.at[pl.ds(r0, C2)])

    @pl.when(cid == 0)
    def _wd():
        pltpu.sync_copy(sden.at[pl.ds(sid * ROWS_PER_TILE, ROWS_PER_TILE)],
                        den_hbm.at[pl.ds(sid * ROWS_PER_TILE, ROWS_PER_TILE)])


def _pass2(q0, q1, packed_p, w):
    half = jax.ShapeDtypeStruct((NPAD, HF), jnp.float32)
    kfn = pl.kernel(
        _pass2_body,
        out_type=(half, half, jax.ShapeDtypeStruct((NPAD,), jnp.float32)),
        mesh=_mesh,
        compiler_params=_sc_params,
        scratch_types=(
            [pltpu.VMEM((E_TILE2,), jnp.int32),
             pltpu.VMEM((E_TILE2,), jnp.float32)]
            + [pltpu.VMEM((C2,), jnp.int32)] * 8
            + [pltpu.VMEM((C2, HF), jnp.float32)] * 4
            + [pltpu.VMEM((ROWS_PER_TILE,), jnp.float32),
               pltpu.VMEM_SHARED((NPAD, HF), jnp.float32),
               pltpu.VMEM_SHARED((NPAD,), jnp.float32)]
            + [pltpu.SemaphoreType.DMA] * 8
        ),
    )
    return kfn(q0, q1, packed_p, w)


def _div_body(n0_ref, n1_ref, d_ref, o_ref):
    d = d_ref[...] + 1e-16
    o_ref[:, :HF] = n0_ref[...] / d
    o_ref[:, HF:] = n1_ref[...] / d


def _divide(num0, num1, den2d):
    blk = 1024
    return pl.pallas_call(
        _div_body,
        grid=(NPAD // blk,),
        in_specs=[
            pl.BlockSpec((blk, HF), lambda i: (i, 0)),
            pl.BlockSpec((blk, HF), lambda i: (i, 0)),
            pl.BlockSpec((blk, 1), lambda i: (i, 0)),
        ],
        out_specs=pl.BlockSpec((blk, F), lambda i: (i, 0)),
        out_shape=jax.ShapeDtypeStruct((NPAD, F), jnp.float32),
    )(num0, num1, den2d)


def kernel(x, Wq, Wk, edge_index):
    row = edge_index[0].astype(jnp.int32)
    col = edge_index[1].astype(jnp.int32)
    pad = jnp.full((EPAD - E,), N, dtype=jnp.int32)
    row_p = jnp.concatenate([row, pad])
    col_p = jnp.concatenate([col, pad])
    packed_p = (row_p << PACK_SHIFT) | col_p
    x_pad = jnp.pad(x, ((0, NPAD - N), (0, 0)))

    qw, kw, q0w, q1w = _proj(x_pad, Wq, Wk)
    w = _pass1(qw, kw, packed_p)
    num0, num1, den = _pass2(q0w, q1w, packed_p, w)
    out = _divide(num0, num1, den.reshape(NPAD, 1))
    return out[:N]


# speedup vs baseline: 1.0659x; 1.0659x over previous
import dataclasses
import functools
import math

import jax
import jax.numpy as jnp
from jax import lax
from jax.experimental import pallas as pl
from jax.experimental.pallas import tpu as pltpu
from jax.experimental.pallas import tpu_sc as plsc

N = 10000
E = 160000
F = 256
HF = 128
SCALE = math.sqrt(F)

NPAD = 10240
EPAD = 163840
NC = 2
NS = 16
PACK_SHIFT = 14

E_TILE1 = EPAD // (NC * NS)
C1 = 64
P1_CHUNKS = E_TILE1 // C1

E_TILE2 = EPAD // NS
C2 = 32
P2_CHUNKS = E_TILE2 // C2
P2_SLOTS = 5
P2_AHEAD = 3
ROWS_PER_TILE = NPAD // NS

_mesh = plsc.VectorSubcoreMesh(core_axis_name="c", subcore_axis_name="s")

_sc_params = pltpu.CompilerParams()
if "needs_layout_passes" in pltpu.CompilerParams.__dataclass_fields__:
    _sc_params = dataclasses.replace(_sc_params, needs_layout_passes=False)


def _unpack_idx(packed_all, off, ridx, cidx, n):
    @pl.loop(0, n // 16)
    def _grp(g):
        p = packed_all[pl.ds(off + g * 16, 16)]
        ridx[pl.ds(g * 16, 16)] = lax.shift_right_logical(p, PACK_SHIFT)
        cidx[pl.ds(g * 16, 16)] = lax.bitwise_and(p, (1 << PACK_SHIFT) - 1)


def _pack_words(lo_f32, hi_f32):
    lo = lax.bitcast_convert_type(lo_f32.astype(jnp.bfloat16), jnp.uint16)
    hi = lax.bitcast_convert_type(hi_f32.astype(jnp.bfloat16), jnp.uint16)
    return lo.astype(jnp.int32) | (hi.astype(jnp.int32) << 16)


def _proj_body(x_ref, wq_ref, wk_ref, qw_ref, kw_ref, q0w_ref, q1w_ref):
    xb = x_ref[...]
    dn = (((1,), (1,)), ((), ()))
    q = lax.dot_general(xb, wq_ref[...], dn,
                        preferred_element_type=jnp.float32,
                        precision=lax.Precision.HIGHEST)
    k = lax.dot_general(xb, wk_ref[...], dn,
                        preferred_element_type=jnp.float32,
                        precision=lax.Precision.HIGHEST)
    qw_ref[...] = _pack_words(q[:, :HF], q[:, HF:])
    kw_ref[...] = _pack_words(k[:, :HF], k[:, HF:])
    q0w_ref[...] = q[:, :HF]
    q1w_ref[...] = q[:, HF:]


def _proj(x_pad, Wq, Wk):
    blk = 1024
    wordt = jax.ShapeDtypeStruct((NPAD, HF), jnp.int32)
    half = jax.ShapeDtypeStruct((NPAD, HF), jnp.float32)
    return pl.pallas_call(
        _proj_body,
        grid=(NPAD // blk,),
        in_specs=[
            pl.BlockSpec((blk, F), lambda i: (i, 0)),
            pl.BlockSpec((F, F), lambda i: (0, 0)),
            pl.BlockSpec((F, F), lambda i: (0, 0)),
        ],
        out_specs=[
            pl.BlockSpec((blk, HF), lambda i: (i, 0)),
            pl.BlockSpec((blk, HF), lambda i: (i, 0)),
            pl.BlockSpec((blk, HF), lambda i: (i, 0)),
            pl.BlockSpec((blk, HF), lambda i: (i, 0)),
        ],
        out_shape=[wordt, wordt, half, half],
    )(x_pad, Wq, Wk)


def _p1_start(q_hbm, k_hbm, packed, off, ridx, cidx, qv, kv, sem):
    _unpack_idx(packed, off, ridx, cidx, C1)
    pltpu.async_copy(q_hbm.at[ridx], qv, sem)
    pltpu.async_copy(k_hbm.at[cidx], kv, sem)


def _p1_wait(q_hbm, k_hbm, ridx, cidx, qv, kv, sem):
    pltpu.make_async_copy(q_hbm.at[ridx], qv, sem).wait()
    pltpu.make_async_copy(k_hbm.at[cidx], kv, sem).wait()


def _bf16_pair(ref, e, t):
    words = ref[e, pl.ds(t * 16, 16)]
    hi = plsc.bitcast(lax.bitwise_and(words, jnp.int32(-65536)), jnp.float32)
    lo = plsc.bitcast(lax.shift_left(words, 16), jnp.float32)
    return lo, hi


def _p1_compute(qv, kv, wv_all, off):
    lane = lax.iota(jnp.int32, 16)

    @pl.loop(0, C1 // 16)
    def _grp(g):
        dots = jnp.zeros((16,), jnp.float32)
        for j in range(16):
            e = g * 16 + j
            acc0 = jnp.zeros((16,), jnp.float32)
            acc1 = jnp.zeros((16,), jnp.float32)
            for t in range(F // 32):
                qlo, qhi = _bf16_pair(qv, e, t)
                klo, khi = _bf16_pair(kv, e, t)
                acc0 = acc0 + qlo * klo
                acc1 = acc1 + qhi * khi
            dots = jnp.where(lane == j, jnp.sum(acc0 + acc1), dots)
        wv_all[pl.ds(off + g * 16, 16)] = jnp.exp(dots * (1.0 / SCALE))


P1_SLOTS = 5
P1_AHEAD = 3


def _pass1_body(q_hbm, k_hbm, packed_hbm, w_hbm,
                packed, wv_all,
                r0, c0, r1, c1, r2, c2x, r3, c3, r4, c4,
                qv0, kv0, qv1, kv1, qv2, kv2, qv3, kv3, qv4, kv4,
                s0, s1, s2, s3, s4):
    wid = lax.axis_index("s") * NC + lax.axis_index("c")
    base = wid * E_TILE1
    ridx = [r0, r1, r2, r3, r4]
    cidx = [c0, c1, c2x, c3, c4]
    qv = [qv0, qv1, qv2, qv3, qv4]
    kv = [kv0, kv1, kv2, kv3, kv4]
    sem = [s0, s1, s2, s3, s4]

    pltpu.sync_copy(packed_hbm.at[pl.ds(base, E_TILE1)], packed)
    for p in range(P1_AHEAD):
        _p1_start(q_hbm, k_hbm, packed, p * C1,
                  ridx[p], cidx[p], qv[p], kv[p], sem[p])

    @pl.loop(0, P1_CHUNKS // P1_SLOTS)
    def _it(i):
        cb = P1_SLOTS * i
        for p in range(P1_SLOTS):
            c = cb + p
            _p1_wait(q_hbm, k_hbm, ridx[p], cidx[p], qv[p], kv[p], sem[p])

            s2_ = (p + P1_AHEAD) % P1_SLOTS

            @pl.when(c + P1_AHEAD < P1_CHUNKS)
            def _pf():
                _p1_start(q_hbm, k_hbm, packed, (c + P1_AHEAD) * C1,
                          ridx[s2_], cidx[s2_], qv[s2_], kv[s2_], sem[s2_])

            _p1_compute(qv[p], kv[p], wv_all, c * C1)

    pltpu.sync_copy(wv_all, w_hbm.at[pl.ds(base, E_TILE1)])


def _pass1(q, k, packed_p):
    kfn = pl.kernel(
        _pass1_body,
        out_type=jax.ShapeDtypeStruct((EPAD,), jnp.float32),
        mesh=_mesh,
        compiler_params=_sc_params,
        scratch_types=(
            [pltpu.VMEM((E_TILE1,), jnp.int32),
             pltpu.VMEM((E_TILE1,), jnp.float32)]
            + [pltpu.VMEM((C1,), jnp.int32)] * (2 * P1_SLOTS)
            + [pltpu.VMEM((C1, HF), jnp.int32)] * (2 * P1_SLOTS)
            + [pltpu.SemaphoreType.DMA] * P1_SLOTS
        ),
    )
    return kfn(q, k, packed_p)


def _p2_scale(qv, wv_all, off):
    @pl.loop(0, C2 // 16)
    def _grp(g):
        ws = wv_all[pl.ds(off + g * 16, 16)]
        for j in range(16):
            e = g * 16 + j
            we = ws[j]
            for t in range(HF // 16):
                qv[e, pl.ds(t * 16, 16)] = qv[e, pl.ds(t * 16, 16)] * we


def _p2_scatter_start(qv, wv_all, off, cidx, snum, sden, sem):
    pltpu.async_copy(qv, snum.at[cidx], sem, add=True)
    pltpu.async_copy(wv_all.at[pl.ds(off, C2)], sden.at[cidx], sem, add=True)


def _p2_scatter_wait(qv, wv_all, off, cidx, snum, sden, sem):
    pltpu.make_async_copy(qv, snum.at[cidx], sem).wait()
    pltpu.make_async_copy(wv_all.at[pl.ds(off, C2)], sden.at[cidx], sem).wait()


def _pass2_body(q0_hbm, q1_hbm, packed_hbm, w_hbm,
                num0_hbm, num1_hbm, den_hbm,
                packed, wv_all,
                ra, ca, rb, cb, rc, cc, rd, cd, re, ce,
                qva, qvb, qvc, qvd, qve, zd,
                snum, sden,
                gsa, gsb, gsc, gsd, gse, ssa, ssb, ssc, ssd, sse):
    cid = lax.axis_index("c")
    sid = lax.axis_index("s")
    base = sid * E_TILE2
    ridx = [ra, rb, rc, rd, re]
    cidx = [ca, cb, cc, cd, ce]
    qv = [qva, qvb, qvc, qvd, qve]
    gsem = [gsa, gsb, gsc, gsd, gse]
    ssem = [ssa, ssb, ssc, ssd, sse]

    pltpu.sync_copy(packed_hbm.at[pl.ds(base, E_TILE2)], packed)
    pltpu.sync_copy(w_hbm.at[pl.ds(base, E_TILE2)], wv_all)

    @pl.loop(0, C2)
    def _z(r):
        for t in range(HF // 16):
            qva[r, pl.ds(t * 16, 16)] = jnp.zeros((16,), jnp.float32)

    @pl.loop(0, ROWS_PER_TILE // 16)
    def _zd(i):
        zd[pl.ds(i * 16, 16)] = jnp.zeros((16,), jnp.float32)

    @pl.loop(0, ROWS_PER_TILE // C2)
    def _zs(b):
        pltpu.sync_copy(qva, snum.at[pl.ds(sid * ROWS_PER_TILE + b * C2, C2)])

    pltpu.sync_copy(zd, sden.at[pl.ds(sid * ROWS_PER_TILE, ROWS_PER_TILE)])
    plsc.subcore_barrier()

    nch = P2_CHUNKS

    def run(q_hbm):
        for p in range(P2_AHEAD):
            _unpack_idx(packed, p * C2, ridx[p], cidx[p], C2)
            pltpu.async_copy(q_hbm.at[ridx[p]], qv[p], gsem[p])

        @pl.loop(0, nch // P2_SLOTS)
        def _it(i):
            cbase = P2_SLOTS * i
            for p in range(P2_SLOTS):
                c = cbase + p
                off = c * C2
                pltpu.make_async_copy(q_hbm.at[ridx[p]], qv[p], gsem[p]).wait()
                _p2_scale(qv[p], wv_all, off)
                _p2_scatter_start(qv[p], wv_all, off, cidx[p], snum, sden,
                                  ssem[p])
                s2 = (p + P2_AHEAD) % P2_SLOTS
                back = P2_SLOTS - P2_AHEAD

                @pl.when(c + P2_AHEAD < nch)
                def _pf():
                    @pl.when(c >= back)
                    def _ws():
                        _p2_scatter_wait(qv[s2], wv_all, (c - back) * C2,
                                         cidx[s2], snum, sden, ssem[s2])

                    _unpack_idx(packed, (c + P2_AHEAD) * C2,
                                ridx[s2], cidx[s2], C2)
                    pltpu.async_copy(q_hbm.at[ridx[s2]], qv[s2], gsem[s2])

        for p in range(P2_SLOTS):
            _p2_scatter_wait(qv[p], wv_all, (nch - P2_SLOTS + p) * C2, cidx[p],
                             snum, sden, ssem[p])

    @pl.when(cid == 0)
    def _c0():
        run(q0_hbm)

    @pl.when(cid == 1)
    def _c1():
        run(q1_hbm)

    plsc.subcore_barrier()

    @pl.loop(0, ROWS_PER_TILE // C2)
    def _wb(b):
        r0 = sid * ROWS_PER_TILE + b * C2

        @pl.when(cid == 0)
        def _w0():
            pltpu.sync_copy(snum.at[pl.ds(r0, C2)], num0_hbm.at[pl.ds(r0, C2)])

        @pl.when(cid == 1)
        def _w1():
            pltpu.sync_copy(snum.at[pl.ds(r0, C2)], num1_hbm.at[pl.ds(r0, C2)])

    @pl.when(cid == 0)
    def _wd():
        pltpu.sync_copy(sden.at[pl.ds(sid * ROWS_PER_TILE, ROWS_PER_TILE)],
                        den_hbm.at[pl.ds(sid * ROWS_PER_TILE, ROWS_PER_TILE)])


def _pass2(q0, q1, packed_p, w):
    half = jax.ShapeDtypeStruct((NPAD, HF), jnp.float32)
    kfn = pl.kernel(
        _pass2_body,
        out_type=(half, half, jax.ShapeDtypeStruct((NPAD,), jnp.float32)),
        mesh=_mesh,
        compiler_params=_sc_params,
        scratch_types=(
            [pltpu.VMEM((E_TILE2,), jnp.int32),
             pltpu.VMEM((E_TILE2,), jnp.float32)]
            + [pltpu.VMEM((C2,), jnp.int32)] * (2 * P2_SLOTS)
            + [pltpu.VMEM((C2, HF), jnp.float32)] * P2_SLOTS
            + [pltpu.VMEM((ROWS_PER_TILE,), jnp.float32),
               pltpu.VMEM_SHARED((NPAD, HF), jnp.float32),
               pltpu.VMEM_SHARED((NPAD,), jnp.float32)]
            + [pltpu.SemaphoreType.DMA] * (2 * P2_SLOTS)
        ),
    )
    return kfn(q0, q1, packed_p, w)


def _div_body(n0_ref, n1_ref, d_ref, o_ref):
    d = d_ref[...] + 1e-16
    o_ref[:, :HF] = n0_ref[...] / d
    o_ref[:, HF:] = n1_ref[...] / d


def _divide(num0, num1, den2d):
    blk = 1024
    return pl.pallas_call(
        _div_body,
        grid=(NPAD // blk,),
        in_specs=[
            pl.BlockSpec((blk, HF), lambda i: (i, 0)),
            pl.BlockSpec((blk, HF), lambda i: (i, 0)),
            pl.BlockSpec((blk, 1), lambda i: (i, 0)),
        ],
        out_specs=pl.BlockSpec((blk, F), lambda i: (i, 0)),
        out_shape=jax.ShapeDtypeStruct((NPAD, F), jnp.float32),
    )(num0, num1, den2d)


def kernel(x, Wq, Wk, edge_index):
    row = edge_index[0].astype(jnp.int32)
    col = edge_index[1].astype(jnp.int32)
    pad = jnp.full((EPAD - E,), N, dtype=jnp.int32)
    row_p = jnp.concatenate([row, pad])
    col_p = jnp.concatenate([col, pad])
    packed_p = (row_p << PACK_SHIFT) | col_p
    x_pad = jnp.pad(x, ((0, NPAD - N), (0, 0)))

    qw, kw, q0w, q1w = _proj(x_pad, Wq, Wk)
    w = _pass1(qw, kw, packed_p)
    num0, num1, den = _pass2(q0w, q1w, packed_p, w)
    out = _divide(num0, num1, den.reshape(NPAD, 1))
    return out[:N]
